# Initial kernel scaffold; baseline (speedup 1.0000x reference)
import jax
import jax.numpy as jnp
from jax.experimental import pallas as pl


def kernel(x, x1, x2):
    raise NotImplementedError("write your pallas kernel here")


# speedup vs baseline: 1.8118x; 1.8118x over previous
import functools

import jax
import jax.numpy as jnp
from jax import lax
from jax.experimental import pallas as pl
from jax.experimental.pallas import tpu as pltpu
from jax.experimental.pallas import tpu_sc as plsc


def _sc_build(s_len, n0, n1, d0, d1, nc, ns):
    nw = nc * ns
    j_per_w = n1 // nw
    bc_rows = n0 // 2

    mesh = plsc.VectorSubcoreMesh(core_axis_name="c", subcore_axis_name="s")

    @functools.partial(
        pl.kernel,
        out_type=jax.ShapeDtypeStruct((s_len, d0 + d1), jnp.float32),
        mesh=mesh,
        scratch_types=[
            pltpu.VMEM((n0, d0), jnp.float32),
            pltpu.VMEM((bc_rows, d1), jnp.float32),
            pltpu.VMEM((bc_rows,), jnp.int32),
            pltpu.SemaphoreType.DMA,
        ],
    )
    def body(x1_hbm, x2_hbm, out_hbm, x1_v, bc_v, idx_v, sem):
        wid = lax.axis_index("s") * nc + lax.axis_index("c")
        pltpu.sync_copy(x1_hbm, x1_v)
        for t in range(j_per_w):
            j = wid * j_per_w + t
            jvec = jnp.full((16,), j, jnp.int32)
            for q in range(bc_rows // 16):
                idx_v[pl.ds(q * 16, 16)] = jvec
            pltpu.async_copy(x2_hbm.at[idx_v], bc_v, sem).wait()
            base = j * n0
            pltpu.sync_copy(x1_v, out_hbm.at[pl.ds(base, n0), pl.ds(0, d0)])
            pltpu.sync_copy(bc_v, out_hbm.at[pl.ds(base, bc_rows), pl.ds(d0, d1)])
            pltpu.sync_copy(
                bc_v, out_hbm.at[pl.ds(base + bc_rows, bc_rows), pl.ds(d0, d1)]
            )

    return body


def kernel(x, x1, x2):
    s_len = x.shape[1]
    n0, d0 = x1.shape
    n1, d1 = x2.shape
    info = plsc.get_sparse_core_info()
    build = _sc_build(s_len, n0, n1, d0, d1, info.num_cores, info.num_subcores)
    out = build(x1, x2)
    return out.astype(x.dtype)[None, :, :]
